# baseline (speedup 1.0000x reference)
import functools

import jax
import jax.numpy as jnp
from jax import lax
from jax.experimental import pallas as pl
from jax.experimental.pallas import tpu as pltpu
from jax.experimental.pallas import tpu_sc as plsc

B = 4
H = 64
W = 64
N = H * W
K = 20
LARGE = 1e18
EPS = 1e-5


def _roll(h, shift):
    sh = shift % N
    if sh == 0:
        return h
    return jnp.concatenate([h[:, N - sh:], h[:, :N - sh]], axis=1)


def _shift_mask(e, f):
    l = lax.broadcasted_iota(jnp.int32, (1, N), 1)
    y = l // W
    x = l % W
    return ((y + e >= 0) & (y + e < H) & (x + f >= 0) & (x + f < W))


def _conv_patches(h, ksize, pad, rows):
    parts = []
    for e in range(-pad, ksize - pad):
        for f in range(-pad, ksize - pad):
            shifted = _roll(h, -(e * W + f))
            m = _shift_mask(e, f)
            parts.append(jnp.where(m, shifted, 0.0) if rows == 1
                         else jnp.where(m, shifted, 0.0))
    return jnp.concatenate(parts, axis=0)


def _backbone_body(x_ref, w1_ref, b1_ref, w2_ref, b2_ref, w3_ref, b3_ref,
                   w4_ref, b4_ref, h_ref, rank_ref, s_ref):
    x = x_ref[0]
    p1 = _conv_patches(x, 5, 2, 1)
    h1 = jnp.tanh(jnp.dot(w1_ref[...], p1,
                          preferred_element_type=jnp.float32) + b1_ref[...])
    p2 = _conv_patches(h1, 3, 1, 64)
    h2 = jnp.tanh(jnp.dot(w2_ref[...], p2,
                          preferred_element_type=jnp.float32) + b2_ref[...])
    p3 = _conv_patches(h2, 3, 1, 32)
    h3 = jnp.tanh(jnp.dot(w3_ref[...], p3,
                          preferred_element_type=jnp.float32) + b3_ref[...])
    p4 = _conv_patches(h3, 3, 1, 16)
    v = jnp.dot(w4_ref[...], p4,
                preferred_element_type=jnp.float32) + b4_ref[...]
    h_ref[0] = v

    v_col = jnp.transpose(v)
    n_iota = lax.broadcasted_iota(jnp.int32, (1, N), 1)
    JCH = 512
    ones_row = jnp.ones((1, JCH), jnp.float32)
    rank_f = jnp.zeros((1, N), jnp.float32)
    for cj in range(N // JCH):
        vj = v_col[cj * JCH:(cj + 1) * JCH, :]
        j_iota = cj * JCH + lax.broadcasted_iota(jnp.int32, (JCH, 1), 0)
        lt = vj < v
        tie = (vj == v) & (j_iota < n_iota)
        cmpf = jnp.where(lt | tie, 1.0, 0.0)
        rank_f = rank_f + lax.dot_general(
            ones_row, cmpf, (((1,), (0,)), ((), ())),
            precision=lax.Precision.HIGHEST,
            preferred_element_type=jnp.float32)
    rank_row = rank_f.astype(jnp.int32)
    rank_ref[0] = rank_row
    rank_col = jnp.transpose(rank_row)

    SCH = 256
    s_chunks = []
    for ci in range(N // SCH):
        r_iota = ci * SCH + lax.broadcasted_iota(jnp.int32, (N, SCH), 1)
        eqf = jnp.where(rank_col == r_iota, 1.0, 0.0)
        s_chunks.append(lax.dot_general(
            v, eqf, (((1,), (0,)), ((), ())),
            precision=lax.Precision.HIGHEST,
            preferred_element_type=jnp.float32))
    s_ref[0] = jnp.concatenate(s_chunks, axis=1)


def _backbone(xr, w1m, b1, w2m, b2, w3m, b3, w4m, b4):
    full = lambda shape: pl.BlockSpec(shape, lambda b: (0, 0))
    return pl.pallas_call(
        _backbone_body,
        grid=(B,),
        in_specs=[
            pl.BlockSpec((1, 1, N), lambda b: (b, 0, 0)),
            full((64, 25)), full((64, 1)),
            full((32, 576)), full((32, 1)),
            full((16, 288)), full((16, 1)),
            full((1, 144)), full((1, 1)),
        ],
        out_specs=[
            pl.BlockSpec((1, 1, N), lambda b: (b, 0, 0)),
            pl.BlockSpec((1, 1, N), lambda b: (b, 0, 0)),
            pl.BlockSpec((1, 1, N), lambda b: (b, 0, 0)),
        ],
        out_shape=[
            jax.ShapeDtypeStruct((B, 1, N), jnp.float32),
            jax.ShapeDtypeStruct((B, 1, N), jnp.int32),
            jax.ShapeDtypeStruct((B, 1, N), jnp.float32),
        ],
    )(xr.reshape(B, 1, N), w1m, b1, w2m, b2, w3m, b3, w4m, b4)


def _edge_body(s_ref, h_ref, *refs):
    gw = [refs[i * 8:(i + 1) * 8] for i in range(3)]
    sig_ref, res_ref = refs[24], refs[25]

    lane = lax.broadcasted_iota(jnp.int32, (1, N), 1)
    M = float(B * N * K)

    x2s, sels, x1s = [], [], []
    Sx2 = Sx22 = Sx12 = Sv = Svv = 0.0
    for b in range(B):
        srow = s_ref[b:b + 1, :]
        dls, drs = [], []
        for i in range(K):
            o = i + 1
            sl = jnp.where(lane >= o, _roll(srow, o), -LARGE)
            sr = jnp.where(lane < N - o, _roll(srow, -o), LARGE)
            dls.append(srow - sl)
            drs.append(sr - srow)
        sel_rows = []
        for i in range(K):
            sel_rows.append((dls[i] <= drs[K - 1 - i]).astype(jnp.float32))
        for i in range(K):
            sel_rows.append((drs[i] < dls[K - 1 - i]).astype(jnp.float32))
        sel = jnp.concatenate(sel_rows, axis=0)
        x2 = jnp.concatenate([-d for d in dls] + drs, axis=0)
        x1 = jnp.broadcast_to(srow, (2 * K, N))
        selF = sel
        Sx2 = Sx2 + jnp.sum(selF * x2)
        Sx22 = Sx22 + jnp.sum(selF * x2 * x2)
        Sx12 = Sx12 + jnp.sum(selF * x1 * x2)
        Sv = Sv + jnp.sum(srow)
        Svv = Svv + jnp.sum(srow * srow)
        x2s.append(x2)
        sels.append(sel)
        x1s.append(x1)
        sig_ref[b:b + 1, :] = 1.0 / (1.0 + jnp.exp(-h_ref[b:b + 1, :]))

    mu1 = K * Sv / M
    mu2 = Sx2 / M
    c11 = K * Svv / M - mu1 * mu1
    c22 = Sx22 / M - mu2 * mu2
    c12 = Sx12 / M - mu1 * mu2

    for g in range(3):
        w1, b1, g1, be1, w2, b2, g2, be2 = gw[g]
        S2 = 0.0
        S22 = 0.0
        maxs, mins = [], []
        for b in range(B):
            x1, x2, sel = x1s[b], x2s[b], sels[b]
            selF = sel

            def chan_body(c, acc):
                w0 = w1[pl.ds(c, 1), 0:1]
                wA = w1[pl.ds(c, 1), 1:2]
                m = w0 * mu1 + wA * mu2 + b1[pl.ds(c, 1), 0:1]
                var = w0 * w0 * c11 + wA * wA * c22 + 2.0 * w0 * wA * c12
                inv = g1[pl.ds(c, 1), 0:1] * lax.rsqrt(var + EPS)
                pre1 = w0 * x1 + wA * x2
                h1 = jnp.maximum((pre1 - m) * inv + be1[pl.ds(c, 1), 0:1],
                                 0.0)
                return acc + w2[pl.ds(c, 1), 0:1] * h1

            pre2 = lax.fori_loop(
                0, 16, chan_body,
                jnp.zeros((2 * K, N), jnp.float32) + b2[0:1, 0:1])
            S2 = S2 + jnp.sum(selF * pre2)
            S22 = S22 + jnp.sum(selF * pre2 * pre2)
            maxs.append(jnp.max(jnp.where(sel > 0.5, pre2, -LARGE), axis=0,
                                keepdims=True))
            mins.append(jnp.min(jnp.where(sel > 0.5, pre2, LARGE), axis=0,
                                keepdims=True))
        mu = S2 / M
        var = S22 / M - mu * mu
        inv2 = g2[0:1, 0:1] * lax.rsqrt(var + EPS)
        be2v = be2[0:1, 0:1]
        for b in range(B):
            a = jnp.maximum((maxs[b] - mu) * inv2 + be2v, 0.0)
            c2 = jnp.maximum((mins[b] - mu) * inv2 + be2v, 0.0)
            val = jnp.maximum(a, c2)
            res_ref[g * B + b:g * B + b + 1, :] = 1.0 / (1.0 + jnp.exp(-val))


def _edge(s, h, gcn_params):
    flat = []
    for p in gcn_params:
        flat.extend(p)
    full = lambda a: pl.BlockSpec(a.shape, lambda: tuple(0 for _ in a.shape))
    return pl.pallas_call(
        _edge_body,
        in_specs=[full(s), full(h)] + [full(a) for a in flat],
        out_specs=[
            pl.BlockSpec((B, N), lambda: (0, 0)),
            pl.BlockSpec((3 * B, N), lambda: (0, 0)),
        ],
        out_shape=[
            jax.ShapeDtypeStruct((B, N), jnp.float32),
            jax.ShapeDtypeStruct((3 * B, N), jnp.float32),
        ],
    )(s, h, *flat)


def _sc_gather(ress, rank3):
    mesh = plsc.VectorSubcoreMesh(core_axis_name="c", subcore_axis_name="s")

    @functools.partial(
        pl.kernel,
        out_type=jax.ShapeDtypeStruct((12, 32, 128), jnp.float32),
        mesh=mesh,
        scratch_types=[
            pltpu.VMEM((N,), jnp.float32),
            pltpu.VMEM((128,), jnp.int32),
            pltpu.VMEM((128,), jnp.float32),
        ],
        compiler_params=pltpu.CompilerParams(needs_layout_passes=False),
    )
    def k(ress_hbm, rank_hbm, out_hbm, table_v, idx_v, buf_v):
        wid = lax.axis_index("s") * 2 + lax.axis_index("c")
        for b in range(B):
            pltpu.sync_copy(rank_hbm.at[b, wid], idx_v)
            for g in range(3):
                row = g * B + b
                pltpu.sync_copy(ress_hbm.at[row], table_v)
                for j in range(8):
                    idx = idx_v[pl.ds(j * 16, 16)]
                    buf_v[pl.ds(j * 16, 16)] = plsc.load_gather(
                        table_v, [idx])
                pltpu.sync_copy(buf_v, out_hbm.at[row, wid])

    return k(ress, rank3)


def kernel(x, conv1_w, conv1_b, conv2_w, conv2_b, conv3_w, conv3_b, conv4_w,
           conv4_b,
           gcn1_w1, gcn1_b1, gcn1_g1, gcn1_be1, gcn1_w2, gcn1_b2, gcn1_g2,
           gcn1_be2,
           gcn2_w1, gcn2_b1, gcn2_g1, gcn2_be1, gcn2_w2, gcn2_b2, gcn2_g2,
           gcn2_be2,
           gcn3_w1, gcn3_b1, gcn3_g1, gcn3_be1, gcn3_w2, gcn3_b2, gcn3_g2,
           gcn3_be2):
    xr = x.reshape(B, N)
    w1m = conv1_w.reshape(64, 25)
    w2m = jnp.transpose(conv2_w, (0, 2, 3, 1)).reshape(32, 576)
    w3m = jnp.transpose(conv3_w, (0, 2, 3, 1)).reshape(16, 288)
    w4m = jnp.transpose(conv4_w, (0, 2, 3, 1)).reshape(1, 144)
    h, rank, s = _backbone(xr, w1m, conv1_b.reshape(64, 1),
                           w2m, conv2_b.reshape(32, 1),
                           w3m, conv3_b.reshape(16, 1),
                           w4m, conv4_b.reshape(1, 1))
    h = h.reshape(B, N)
    s = s.reshape(B, N)

    gcn_params = []
    for (w1, b1, g1, be1, w2, b2, g2, be2) in (
            (gcn1_w1, gcn1_b1, gcn1_g1, gcn1_be1, gcn1_w2, gcn1_b2, gcn1_g2,
             gcn1_be2),
            (gcn2_w1, gcn2_b1, gcn2_g1, gcn2_be1, gcn2_w2, gcn2_b2, gcn2_g2,
             gcn2_be2),
            (gcn3_w1, gcn3_b1, gcn3_g1, gcn3_be1, gcn3_w2, gcn3_b2, gcn3_g2,
             gcn3_be2)):
        gcn_params.append((w1.reshape(16, 2), b1.reshape(16, 1),
                           g1.reshape(16, 1), be1.reshape(16, 1),
                           w2.reshape(16, 1), b2.reshape(1, 1),
                           g2.reshape(1, 1), be2.reshape(1, 1)))

    sigh, ress = _edge(s, h, gcn_params)

    res = _sc_gather(ress, rank.reshape(B, 32, 128)).reshape(3, B, H, W)

    ch0 = sigh.reshape(B, H, W)
    p = jnp.stack([ch0, res[0], res[1], res[2]], axis=1)
    p = p.reshape(B, 1, 2, 2, H, W)
    p = jnp.transpose(p, (0, 1, 4, 2, 5, 3))
    return p.reshape(B, 1, 2 * H, 2 * W)


# speedup vs baseline: 1.4180x; 1.4180x over previous
import functools

import jax
import jax.numpy as jnp
from jax import lax
from jax.experimental import pallas as pl
from jax.experimental.pallas import tpu as pltpu
from jax.experimental.pallas import tpu_sc as plsc

B = 4
H = 64
W = 64
N = H * W
K = 20
LARGE = 1e18
EPS = 1e-5


def _roll(h, shift):
    sh = shift % N
    if sh == 0:
        return h
    return jnp.concatenate([h[:, N - sh:], h[:, :N - sh]], axis=1)


def _shift_mask(e, f):
    l = lax.broadcasted_iota(jnp.int32, (1, N), 1)
    y = l // W
    x = l % W
    return ((y + e >= 0) & (y + e < H) & (x + f >= 0) & (x + f < W))


def _conv_patches(h, ksize, pad, rows):
    parts = []
    for e in range(-pad, ksize - pad):
        for f in range(-pad, ksize - pad):
            shifted = _roll(h, -(e * W + f))
            m = _shift_mask(e, f)
            parts.append(jnp.where(m, shifted, 0.0) if rows == 1
                         else jnp.where(m, shifted, 0.0))
    return jnp.concatenate(parts, axis=0)


def _fused_body(x_ref, w1_ref, b1_ref, w2_ref, b2_ref, w3_ref, b3_ref,
                w4_ref, b4_ref, *refs):
    gw = [refs[i * 8:(i + 1) * 8] for i in range(3)]
    rank_ref, sig_ref, res_ref, s_scr, h_scr = refs[24:29]
    step = pl.program_id(0)

    @pl.when(step < B)
    def _backbone():
        x = x_ref[0]
        p1 = _conv_patches(x, 5, 2, 1)
        h1 = jnp.tanh(jnp.dot(w1_ref[...], p1,
                              preferred_element_type=jnp.float32)
                      + b1_ref[...])
        p2 = _conv_patches(h1, 3, 1, 64)
        h2 = jnp.tanh(jnp.dot(w2_ref[...], p2,
                              preferred_element_type=jnp.float32)
                      + b2_ref[...])
        p3 = _conv_patches(h2, 3, 1, 32)
        h3 = jnp.tanh(jnp.dot(w3_ref[...], p3,
                              preferred_element_type=jnp.float32)
                      + b3_ref[...])
        p4 = _conv_patches(h3, 3, 1, 16)
        v = jnp.dot(w4_ref[...], p4,
                    preferred_element_type=jnp.float32) + b4_ref[...]
        h_scr[pl.ds(step, 1), :] = v

        v_col = jnp.transpose(v)
        j_lane = lax.broadcasted_iota(jnp.int32, (1, N), 1)
        CH = 512
        rank_chunks = []
        for ci in range(N // CH):
            vn = v_col[ci * CH:(ci + 1) * CH, :]
            n_iota = ci * CH + lax.broadcasted_iota(jnp.int32, (CH, 1), 0)
            lt = v < vn
            tie = (v == vn) & (j_lane < n_iota)
            cnt = jnp.sum((lt | tie).astype(jnp.int32), axis=1)
            rank_chunks.append(cnt.reshape(CH, 1))
        rank_col = jnp.concatenate(rank_chunks, axis=0)
        rank_ref[0] = jnp.transpose(rank_col)

        SCH = 256
        s_chunks = []
        for ci in range(N // SCH):
            r_iota = ci * SCH + lax.broadcasted_iota(jnp.int32, (N, SCH), 1)
            eq = rank_col == r_iota
            s_chunks.append(jnp.sum(jnp.where(eq, v_col, 0.0), axis=0)
                            .reshape(1, SCH))
        s_scr[pl.ds(step, 1), :] = jnp.concatenate(s_chunks, axis=1)

    @pl.when(step == B)
    def _edge():
        lane = lax.broadcasted_iota(jnp.int32, (1, N), 1)
        M = float(B * N * K)

        x2s, sels, x1s = [], [], []
        Sx2 = Sx22 = Sx12 = Sv = Svv = 0.0
        for b in range(B):
            srow = s_scr[b:b + 1, :]
            dls, drs = [], []
            for i in range(K):
                o = i + 1
                sl = jnp.where(lane >= o, _roll(srow, o), -LARGE)
                sr = jnp.where(lane < N - o, _roll(srow, -o), LARGE)
                dls.append(srow - sl)
                drs.append(sr - srow)
            sel_rows = []
            for i in range(K):
                sel_rows.append((dls[i] <= drs[K - 1 - i])
                                .astype(jnp.float32))
            for i in range(K):
                sel_rows.append((drs[i] < dls[K - 1 - i])
                                .astype(jnp.float32))
            sel = jnp.concatenate(sel_rows, axis=0)
            x2 = jnp.concatenate([-d for d in dls] + drs, axis=0)
            x1 = jnp.broadcast_to(srow, (2 * K, N))
            Sx2 = Sx2 + jnp.sum(sel * x2)
            Sx22 = Sx22 + jnp.sum(sel * x2 * x2)
            Sx12 = Sx12 + jnp.sum(sel * x1 * x2)
            Sv = Sv + jnp.sum(srow)
            Svv = Svv + jnp.sum(srow * srow)
            x2s.append(x2)
            sels.append(sel)
            x1s.append(x1)
            sig_ref[b:b + 1, :] = 1.0 / (1.0 + jnp.exp(-h_scr[b:b + 1, :]))

        mu1 = K * Sv / M
        mu2 = Sx2 / M
        c11 = K * Svv / M - mu1 * mu1
        c22 = Sx22 / M - mu2 * mu2
        c12 = Sx12 / M - mu1 * mu2

        for g in range(3):
            w1, b1, g1, be1, w2, b2, g2, be2 = gw[g]
            S2 = 0.0
            S22 = 0.0
            maxs, mins = [], []
            for b in range(B):
                x1, x2, sel = x1s[b], x2s[b], sels[b]

                def chan_body(c, acc):
                    w0 = w1[pl.ds(c, 1), 0:1]
                    wA = w1[pl.ds(c, 1), 1:2]
                    m = w0 * mu1 + wA * mu2 + b1[pl.ds(c, 1), 0:1]
                    var = (w0 * w0 * c11 + wA * wA * c22
                           + 2.0 * w0 * wA * c12)
                    inv = g1[pl.ds(c, 1), 0:1] * lax.rsqrt(var + EPS)
                    pre1 = w0 * x1 + wA * x2
                    h1 = jnp.maximum(
                        (pre1 - m) * inv + be1[pl.ds(c, 1), 0:1], 0.0)
                    return acc + w2[pl.ds(c, 1), 0:1] * h1

                pre2 = lax.fori_loop(
                    0, 16, chan_body,
                    jnp.zeros((2 * K, N), jnp.float32) + b2[0:1, 0:1])
                S2 = S2 + jnp.sum(sel * pre2)
                S22 = S22 + jnp.sum(sel * pre2 * pre2)
                maxs.append(jnp.max(jnp.where(sel > 0.5, pre2, -LARGE),
                                    axis=0, keepdims=True))
                mins.append(jnp.min(jnp.where(sel > 0.5, pre2, LARGE),
                                    axis=0, keepdims=True))
            mu = S2 / M
            var = S22 / M - mu * mu
            inv2 = g2[0:1, 0:1] * lax.rsqrt(var + EPS)
            be2v = be2[0:1, 0:1]
            for b in range(B):
                a = jnp.maximum((maxs[b] - mu) * inv2 + be2v, 0.0)
                c2 = jnp.maximum((mins[b] - mu) * inv2 + be2v, 0.0)
                val = jnp.maximum(a, c2)
                res_ref[g * B + b:g * B + b + 1, :] = (
                    1.0 / (1.0 + jnp.exp(-val)))


def _fused(xr, w1m, b1, w2m, b2, w3m, b3, w4m, b4, gcn_params):
    flat = []
    for p in gcn_params:
        flat.extend(p)
    full = lambda a: pl.BlockSpec(a.shape, lambda b: (0, 0))
    last = lambda b: (jnp.minimum(b, B - 1), 0, 0)
    return pl.pallas_call(
        _fused_body,
        grid=(B + 1,),
        in_specs=[
            pl.BlockSpec((1, 1, N), last),
            full(w1m), full(b1), full(w2m), full(b2),
            full(w3m), full(b3), full(w4m), full(b4),
        ] + [full(a) for a in flat],
        out_specs=[
            pl.BlockSpec((1, 1, N), last),
            pl.BlockSpec((B, N), lambda b: (0, 0)),
            pl.BlockSpec((3 * B, N), lambda b: (0, 0)),
        ],
        out_shape=[
            jax.ShapeDtypeStruct((B, 1, N), jnp.int32),
            jax.ShapeDtypeStruct((B, N), jnp.float32),
            jax.ShapeDtypeStruct((3 * B, N), jnp.float32),
        ],
        scratch_shapes=[
            pltpu.VMEM((B, N), jnp.float32),
            pltpu.VMEM((B, N), jnp.float32),
        ],
    )(xr.reshape(B, 1, N), w1m, b1, w2m, b2, w3m, b3, w4m, b4, *flat)


def _sc_gather(ress, rank3):
    mesh = plsc.VectorSubcoreMesh(core_axis_name="c", subcore_axis_name="s")

    @functools.partial(
        pl.kernel,
        out_type=jax.ShapeDtypeStruct((12, 32, 128), jnp.float32),
        mesh=mesh,
        scratch_types=[
            pltpu.VMEM((N,), jnp.float32),
            pltpu.VMEM((128,), jnp.int32),
            pltpu.VMEM((128,), jnp.float32),
        ],
        compiler_params=pltpu.CompilerParams(needs_layout_passes=False),
    )
    def k(ress_hbm, rank_hbm, out_hbm, table_v, idx_v, buf_v):
        wid = lax.axis_index("s") * 2 + lax.axis_index("c")
        for b in range(B):
            pltpu.sync_copy(rank_hbm.at[b, wid], idx_v)
            for g in range(3):
                row = g * B + b
                pltpu.sync_copy(ress_hbm.at[row], table_v)
                for j in range(8):
                    idx = idx_v[pl.ds(j * 16, 16)]
                    buf_v[pl.ds(j * 16, 16)] = plsc.load_gather(
                        table_v, [idx])
                pltpu.sync_copy(buf_v, out_hbm.at[row, wid])

    return k(ress, rank3)


def kernel(x, conv1_w, conv1_b, conv2_w, conv2_b, conv3_w, conv3_b, conv4_w,
           conv4_b,
           gcn1_w1, gcn1_b1, gcn1_g1, gcn1_be1, gcn1_w2, gcn1_b2, gcn1_g2,
           gcn1_be2,
           gcn2_w1, gcn2_b1, gcn2_g1, gcn2_be1, gcn2_w2, gcn2_b2, gcn2_g2,
           gcn2_be2,
           gcn3_w1, gcn3_b1, gcn3_g1, gcn3_be1, gcn3_w2, gcn3_b2, gcn3_g2,
           gcn3_be2):
    xr = x.reshape(B, N)
    w1m = conv1_w.reshape(64, 25)
    w2m = jnp.transpose(conv2_w, (0, 2, 3, 1)).reshape(32, 576)
    w3m = jnp.transpose(conv3_w, (0, 2, 3, 1)).reshape(16, 288)
    w4m = jnp.transpose(conv4_w, (0, 2, 3, 1)).reshape(1, 144)

    gcn_params = []
    for (w1, b1, g1, be1, w2, b2, g2, be2) in (
            (gcn1_w1, gcn1_b1, gcn1_g1, gcn1_be1, gcn1_w2, gcn1_b2, gcn1_g2,
             gcn1_be2),
            (gcn2_w1, gcn2_b1, gcn2_g1, gcn2_be1, gcn2_w2, gcn2_b2, gcn2_g2,
             gcn2_be2),
            (gcn3_w1, gcn3_b1, gcn3_g1, gcn3_be1, gcn3_w2, gcn3_b2, gcn3_g2,
             gcn3_be2)):
        gcn_params.append((w1.reshape(16, 2), b1.reshape(16, 1),
                           g1.reshape(16, 1), be1.reshape(16, 1),
                           w2.reshape(16, 1), b2.reshape(1, 1),
                           g2.reshape(1, 1), be2.reshape(1, 1)))

    rank, sigh, ress = _fused(xr, w1m, conv1_b.reshape(64, 1),
                              w2m, conv2_b.reshape(32, 1),
                              w3m, conv3_b.reshape(16, 1),
                              w4m, conv4_b.reshape(1, 1), gcn_params)

    res = _sc_gather(ress, rank.reshape(B, 32, 128)).reshape(3, B, H, W)

    ch0 = sigh.reshape(B, H, W)
    p = jnp.stack([ch0, res[0], res[1], res[2]], axis=1)
    p = p.reshape(B, 1, 2, 2, H, W)
    p = jnp.transpose(p, (0, 1, 4, 2, 5, 3))
    return p.reshape(B, 1, 2 * H, 2 * W)


# speedup vs baseline: 1.9521x; 1.3767x over previous
import functools

import jax
import jax.numpy as jnp
from jax import lax
from jax.experimental import pallas as pl
from jax.experimental.pallas import tpu as pltpu
from jax.experimental.pallas import tpu_sc as plsc

B = 4
H = 64
W = 64
N = H * W
K = 20
LARGE = 1e18
EPS = 1e-5


def _roll(h, shift):
    sh = shift % N
    if sh == 0:
        return h
    return jnp.concatenate([h[:, N - sh:], h[:, :N - sh]], axis=1)


def _shift_mask(e, f):
    l = lax.broadcasted_iota(jnp.int32, (1, N), 1)
    y = l // W
    x = l % W
    return ((y + e >= 0) & (y + e < H) & (x + f >= 0) & (x + f < W))


def _conv_patches(h, ksize, pad, rows):
    parts = []
    for e in range(-pad, ksize - pad):
        for f in range(-pad, ksize - pad):
            shifted = _roll(h, -(e * W + f))
            m = _shift_mask(e, f)
            parts.append(jnp.where(m, shifted, 0.0) if rows == 1
                         else jnp.where(m, shifted, 0.0))
    return jnp.concatenate(parts, axis=0)


def _backbone_body(x_ref, w1_ref, b1_ref, w2_ref, b2_ref, w3_ref, b3_ref,
                   w4_ref, b4_ref, h_ref, rank_ref, s_ref):
    x = x_ref[0]
    p1 = _conv_patches(x, 5, 2, 1)
    h1 = jnp.tanh(jnp.dot(w1_ref[...], p1,
                          preferred_element_type=jnp.float32) + b1_ref[...])
    p2 = _conv_patches(h1, 3, 1, 64)
    h2 = jnp.tanh(jnp.dot(w2_ref[...], p2,
                          preferred_element_type=jnp.float32) + b2_ref[...])
    p3 = _conv_patches(h2, 3, 1, 32)
    h3 = jnp.tanh(jnp.dot(w3_ref[...], p3,
                          preferred_element_type=jnp.float32) + b3_ref[...])
    p4 = _conv_patches(h3, 3, 1, 16)
    v = jnp.dot(w4_ref[...], p4,
                preferred_element_type=jnp.float32) + b4_ref[...]
    h_ref[0] = v

    v_col = jnp.transpose(v)
    j_lane = lax.broadcasted_iota(jnp.int32, (1, N), 1)
    CH = 512
    rank_chunks = []
    for ci in range(N // CH):
        vn = v_col[ci * CH:(ci + 1) * CH, :]
        n_iota = ci * CH + lax.broadcasted_iota(jnp.int32, (CH, 1), 0)
        lt = v < vn
        tie = (v == vn) & (j_lane < n_iota)
        cnt = jnp.sum((lt | tie).astype(jnp.int32), axis=1)
        rank_chunks.append(cnt.reshape(CH, 1))
    rank_col = jnp.concatenate(rank_chunks, axis=0)
    rank_ref[0] = jnp.transpose(rank_col)

    SCH = 256
    s_chunks = []
    for ci in range(N // SCH):
        r_iota = ci * SCH + lax.broadcasted_iota(jnp.int32, (N, SCH), 1)
        eq = rank_col == r_iota
        s_chunks.append(jnp.sum(jnp.where(eq, v_col, 0.0), axis=0)
                        .reshape(1, SCH))
    s_ref[0] = jnp.concatenate(s_chunks, axis=1)


def _backbone(xr, w1m, b1, w2m, b2, w3m, b3, w4m, b4):
    full = lambda shape: pl.BlockSpec(shape, lambda b: (0, 0))
    return pl.pallas_call(
        _backbone_body,
        grid=(B,),
        in_specs=[
            pl.BlockSpec((1, 1, N), lambda b: (b, 0, 0)),
            full((64, 25)), full((64, 1)),
            full((32, 576)), full((32, 1)),
            full((16, 288)), full((16, 1)),
            full((1, 144)), full((1, 1)),
        ],
        out_specs=[
            pl.BlockSpec((1, 1, N), lambda b: (b, 0, 0)),
            pl.BlockSpec((1, 1, N), lambda b: (b, 0, 0)),
            pl.BlockSpec((1, 1, N), lambda b: (b, 0, 0)),
        ],
        out_shape=[
            jax.ShapeDtypeStruct((B, 1, N), jnp.float32),
            jax.ShapeDtypeStruct((B, 1, N), jnp.int32),
            jax.ShapeDtypeStruct((B, 1, N), jnp.float32),
        ],
    )(xr.reshape(B, 1, N), w1m, b1, w2m, b2, w3m, b3, w4m, b4)


def _edge_body(s_ref, h_ref, *refs):
    gw = [refs[i * 8:(i + 1) * 8] for i in range(3)]
    sig_ref, res_ref = refs[24], refs[25]

    lane = lax.broadcasted_iota(jnp.int32, (1, N), 1)
    M = float(B * N * K)

    x2s, sels, x1s = [], [], []
    Sx2 = Sx22 = Sx12 = Sv = Svv = 0.0
    for b in range(B):
        srow = s_ref[b:b + 1, :]
        dls, drs = [], []
        for i in range(K):
            o = i + 1
            sl = jnp.where(lane >= o, _roll(srow, o), -LARGE)
            sr = jnp.where(lane < N - o, _roll(srow, -o), LARGE)
            dls.append(srow - sl)
            drs.append(sr - srow)
        sel_rows = []
        for i in range(K):
            sel_rows.append((dls[i] <= drs[K - 1 - i]).astype(jnp.float32))
        for i in range(K):
            sel_rows.append((drs[i] < dls[K - 1 - i]).astype(jnp.float32))
        sel = jnp.concatenate(sel_rows, axis=0)
        x2 = jnp.concatenate([-d for d in dls] + drs, axis=0)
        x1 = jnp.broadcast_to(srow, (2 * K, N))
        Sx2 = Sx2 + jnp.sum(sel * x2)
        Sx22 = Sx22 + jnp.sum(sel * x2 * x2)
        Sx12 = Sx12 + jnp.sum(sel * x1 * x2)
        Sv = Sv + jnp.sum(srow)
        Svv = Svv + jnp.sum(srow * srow)
        x2s.append(x2)
        sels.append(sel)
        x1s.append(x1)
        sig_ref[b:b + 1, :] = 1.0 / (1.0 + jnp.exp(-h_ref[b:b + 1, :]))

    mu1 = K * Sv / M
    mu2 = Sx2 / M
    c11 = K * Svv / M - mu1 * mu1
    c22 = Sx22 / M - mu2 * mu2
    c12 = Sx12 / M - mu1 * mu2

    for g in range(3):
        w1, b1, g1, be1, w2, b2, g2, be2 = gw[g]
        S2 = 0.0
        S22 = 0.0
        maxs, mins = [], []
        for b in range(B):
            x1, x2, sel = x1s[b], x2s[b], sels[b]

            def chan_body(c, acc):
                w0 = w1[pl.ds(c, 1), 0:1]
                wA = w1[pl.ds(c, 1), 1:2]
                m = w0 * mu1 + wA * mu2 + b1[pl.ds(c, 1), 0:1]
                var = w0 * w0 * c11 + wA * wA * c22 + 2.0 * w0 * wA * c12
                inv = g1[pl.ds(c, 1), 0:1] * lax.rsqrt(var + EPS)
                pre1 = w0 * x1 + wA * x2
                h1 = jnp.maximum((pre1 - m) * inv + be1[pl.ds(c, 1), 0:1],
                                 0.0)
                return acc + w2[pl.ds(c, 1), 0:1] * h1

            pre2 = lax.fori_loop(
                0, 16, chan_body,
                jnp.zeros((2 * K, N), jnp.float32) + b2[0:1, 0:1])
            S2 = S2 + jnp.sum(sel * pre2)
            S22 = S22 + jnp.sum(sel * pre2 * pre2)
            maxs.append(jnp.max(jnp.where(sel > 0.5, pre2, -LARGE), axis=0,
                                keepdims=True))
            mins.append(jnp.min(jnp.where(sel > 0.5, pre2, LARGE), axis=0,
                                keepdims=True))
        mu = S2 / M
        var = S22 / M - mu * mu
        inv2 = g2[0:1, 0:1] * lax.rsqrt(var + EPS)
        be2v = be2[0:1, 0:1]
        for b in range(B):
            a = jnp.maximum((maxs[b] - mu) * inv2 + be2v, 0.0)
            c2 = jnp.maximum((mins[b] - mu) * inv2 + be2v, 0.0)
            val = jnp.maximum(a, c2)
            res_ref[g * B + b:g * B + b + 1, :] = 1.0 / (1.0 + jnp.exp(-val))


def _edge(s, h, gcn_params):
    flat = []
    for p in gcn_params:
        flat.extend(p)
    full = lambda a: pl.BlockSpec(a.shape, lambda: tuple(0 for _ in a.shape))
    return pl.pallas_call(
        _edge_body,
        in_specs=[full(s), full(h)] + [full(a) for a in flat],
        out_specs=[
            pl.BlockSpec((B, N), lambda: (0, 0)),
            pl.BlockSpec((3 * B, N), lambda: (0, 0)),
        ],
        out_shape=[
            jax.ShapeDtypeStruct((B, N), jnp.float32),
            jax.ShapeDtypeStruct((3 * B, N), jnp.float32),
        ],
    )(s, h, *flat)


def _sc_gather(ress, rank3):
    mesh = plsc.VectorSubcoreMesh(core_axis_name="c", subcore_axis_name="s")

    @functools.partial(
        pl.kernel,
        out_type=jax.ShapeDtypeStruct((12, 32, 128), jnp.float32),
        mesh=mesh,
        scratch_types=[
            pltpu.VMEM((N,), jnp.float32),
            pltpu.VMEM((128,), jnp.int32),
            pltpu.VMEM((128,), jnp.float32),
        ],
        compiler_params=pltpu.CompilerParams(needs_layout_passes=False),
    )
    def k(ress_hbm, rank_hbm, out_hbm, table_v, idx_v, buf_v):
        wid = lax.axis_index("s") * 2 + lax.axis_index("c")
        for b in range(B):
            pltpu.sync_copy(rank_hbm.at[b, wid], idx_v)
            for g in range(3):
                row = g * B + b
                pltpu.sync_copy(ress_hbm.at[row], table_v)
                for j in range(8):
                    idx = idx_v[pl.ds(j * 16, 16)]
                    buf_v[pl.ds(j * 16, 16)] = plsc.load_gather(
                        table_v, [idx])
                pltpu.sync_copy(buf_v, out_hbm.at[row, wid])

    return k(ress, rank3)


def kernel(x, conv1_w, conv1_b, conv2_w, conv2_b, conv3_w, conv3_b, conv4_w,
           conv4_b,
           gcn1_w1, gcn1_b1, gcn1_g1, gcn1_be1, gcn1_w2, gcn1_b2, gcn1_g2,
           gcn1_be2,
           gcn2_w1, gcn2_b1, gcn2_g1, gcn2_be1, gcn2_w2, gcn2_b2, gcn2_g2,
           gcn2_be2,
           gcn3_w1, gcn3_b1, gcn3_g1, gcn3_be1, gcn3_w2, gcn3_b2, gcn3_g2,
           gcn3_be2):
    xr = x.reshape(B, N)
    w1m = conv1_w.reshape(64, 25)
    w2m = jnp.transpose(conv2_w, (0, 2, 3, 1)).reshape(32, 576)
    w3m = jnp.transpose(conv3_w, (0, 2, 3, 1)).reshape(16, 288)
    w4m = jnp.transpose(conv4_w, (0, 2, 3, 1)).reshape(1, 144)

    gcn_params = []
    for (w1, b1, g1, be1, w2, b2, g2, be2) in (
            (gcn1_w1, gcn1_b1, gcn1_g1, gcn1_be1, gcn1_w2, gcn1_b2, gcn1_g2,
             gcn1_be2),
            (gcn2_w1, gcn2_b1, gcn2_g1, gcn2_be1, gcn2_w2, gcn2_b2, gcn2_g2,
             gcn2_be2),
            (gcn3_w1, gcn3_b1, gcn3_g1, gcn3_be1, gcn3_w2, gcn3_b2, gcn3_g2,
             gcn3_be2)):
        gcn_params.append((w1.reshape(16, 2), b1.reshape(16, 1),
                           g1.reshape(16, 1), be1.reshape(16, 1),
                           w2.reshape(16, 1), b2.reshape(1, 1),
                           g2.reshape(1, 1), be2.reshape(1, 1)))

    h, rank, s = _backbone(xr, w1m, conv1_b.reshape(64, 1),
                           w2m, conv2_b.reshape(32, 1),
                           w3m, conv3_b.reshape(16, 1),
                           w4m, conv4_b.reshape(1, 1))

    sigh, ress = _edge(s.reshape(B, N), h.reshape(B, N), gcn_params)

    res = _sc_gather(ress, rank.reshape(B, 32, 128)).reshape(3, B, H, W)

    ch0 = sigh.reshape(B, H, W)
    p = jnp.stack([ch0, res[0], res[1], res[2]], axis=1)
    p = p.reshape(B, 1, 2, 2, H, W)
    p = jnp.transpose(p, (0, 1, 4, 2, 5, 3))
    return p.reshape(B, 1, 2 * H, 2 * W)
